# Initial kernel scaffold; baseline (speedup 1.0000x reference)
import jax
import jax.numpy as jnp
from jax.experimental import pallas as pl


def kernel(q, k, queue, write_idx, neg_idx):
    raise NotImplementedError("write your pallas kernel here")


# speedup vs baseline: 23.6663x; 23.6663x over previous
import dataclasses
import functools

import jax
import jax.numpy as jnp
from jax import lax
from jax.experimental import pallas as pl
from jax.experimental.pallas import tpu as pltpu
from jax.experimental.pallas import tpu_sc as plsc

DIM = 128
K = 262144
B = 4096
N_NEG = 512
T = 0.09

NC = 2
NS = 16
NW = NC * NS
QPW = B // NW
BK = 4096
NBLK = K // BK
IDX_CHUNK = 128
NCHUNK = N_NEG // IDX_CHUNK


def _prologue_body(q_ref, k_ref, qn_ref, kn_ref, lpos_ref):
    qv = q_ref[...]
    kv = k_ref[...]
    qn = qv / jnp.maximum(jnp.sqrt(jnp.sum(qv * qv, axis=1, keepdims=True)), 1e-12)
    kn = kv / jnp.maximum(jnp.sqrt(jnp.sum(kv * kv, axis=1, keepdims=True)), 1e-12)
    qn_ref[...] = qn
    kn_ref[...] = kn
    lpos_ref[...] = jnp.sum(qn * kn, axis=1, keepdims=True)


def _prologue(q, k):
    return pl.pallas_call(
        _prologue_body,
        out_shape=(
            jax.ShapeDtypeStruct((B, DIM), jnp.float32),
            jax.ShapeDtypeStruct((B, DIM), jnp.float32),
            jax.ShapeDtypeStruct((B, 1), jnp.float32),
        ),
    )(q, k)


def _table_body(ws_ref, pw_ref, starts_ref, tbl_ref, kn_ref, out_ref):
    i = pl.program_id(0)
    out_ref[...] = tbl_ref[...]
    s = starts_ref[i]
    e = starts_ref[i + 1]

    def wr(j, carry):
        row = ws_ref[j] - i * BK
        src = pw_ref[j]
        out_ref[pl.ds(row, 1), :] = kn_ref[pl.ds(src, 1), :]
        return carry

    lax.fori_loop(s, e, wr, 0)


def _table_scatter(table0, kn, ws, pw, starts):
    return pl.pallas_call(
        _table_body,
        grid=(NBLK,),
        in_specs=[
            pl.BlockSpec(memory_space=pltpu.SMEM),
            pl.BlockSpec(memory_space=pltpu.SMEM),
            pl.BlockSpec(memory_space=pltpu.SMEM),
            pl.BlockSpec((BK, DIM), lambda i: (i, 0)),
            pl.BlockSpec((B, DIM), lambda i: (0, 0)),
        ],
        out_specs=pl.BlockSpec((BK, DIM), lambda i: (i, 0)),
        out_shape=jax.ShapeDtypeStruct((K, DIM), jnp.float32),
    )(ws, pw, starts, table0, kn)


def _untranspose_body(tbl_ref, out_ref):
    out_ref[...] = jnp.transpose(tbl_ref[...])


def _untranspose(table_new):
    return pl.pallas_call(
        _untranspose_body,
        grid=(NBLK,),
        in_specs=[pl.BlockSpec((BK, DIM), lambda i: (i, 0))],
        out_specs=pl.BlockSpec((DIM, BK), lambda i: (0, i)),
        out_shape=jax.ShapeDtypeStruct((DIM, K), jnp.float32),
    )(table_new)


def _lneg_sc_body(table_hbm, qn_hbm, nidx_hbm, out_hbm,
                  idx_v, rows_v, qn_v, out_v, gsem):
    wid = lax.axis_index("s") * NC + lax.axis_index("c")
    base = wid * QPW

    @pl.loop(0, QPW)
    def _q_loop(qi):
        qrow = base + qi
        pltpu.sync_copy(nidx_hbm.at[qrow], idx_v)
        pltpu.sync_copy(qn_hbm.at[qrow], qn_v)
        copies = [
            pltpu.async_copy(
                table_hbm.at[idx_v.at[ck]],
                rows_v.at[pl.ds(ck * IDX_CHUNK, IDX_CHUNK)],
                gsem,
            )
            for ck in range(NCHUNK)
        ]
        for cp in copies:
            cp.wait()
        qvecs = [qn_v[pl.ds(t * 16, 16)] for t in range(DIM // 16)]
        lane = lax.broadcasted_iota(jnp.int32, (16,), 0)
        lane15 = jnp.full((16,), 15, jnp.int32)

        @pl.loop(0, N_NEG, step=16)
        def _j_loop(j0):
            res = jnp.zeros((16,), jnp.float32)
            for l in range(16):
                j = j0 + l
                acc = rows_v[j, pl.ds(0, 16)] * qvecs[0]
                for t in range(1, DIM // 16):
                    acc = acc + rows_v[j, pl.ds(t * 16, 16)] * qvecs[t]
                tot = jnp.cumsum(acc).at[lane15].get(mode="promise_in_bounds")
                res = jnp.where(lane == l, tot, res)
            out_v[pl.ds(j0, 16)] = res

        pltpu.sync_copy(out_v, out_hbm.at[qrow])


def _lneg_sc(table_new, qn, nidx3):
    mesh = plsc.VectorSubcoreMesh(core_axis_name="c", subcore_axis_name="s")
    cp = pltpu.CompilerParams()
    if "needs_layout_passes" in pltpu.CompilerParams.__dataclass_fields__:
        cp = dataclasses.replace(cp, needs_layout_passes=False)
    kern = pl.kernel(
        _lneg_sc_body,
        out_type=jax.ShapeDtypeStruct((B, N_NEG), jnp.float32),
        mesh=mesh,
        compiler_params=cp,
        scratch_types=[
            pltpu.VMEM((NCHUNK, IDX_CHUNK), jnp.int32),
            pltpu.VMEM((N_NEG, DIM), jnp.float32),
            pltpu.VMEM((DIM,), jnp.float32),
            pltpu.VMEM((N_NEG,), jnp.float32),
            pltpu.SemaphoreType.DMA,
        ],
    )
    return kern(table_new, qn, nidx3)


def _epilogue_body(lpos_ref, lneg_ref, out_ref):
    inv_t = jnp.float32(1.0 / T)
    out_ref[...] = jnp.concatenate(
        [lpos_ref[...] * inv_t, lneg_ref[...] * inv_t], axis=1)


def _epilogue(lpos, lneg):
    return pl.pallas_call(
        _epilogue_body,
        out_shape=jax.ShapeDtypeStruct((B, 1 + N_NEG), jnp.float32),
    )(lpos, lneg)


def kernel(q, k, queue, write_idx, neg_idx):
    perm = jnp.argsort(write_idx, stable=True)
    ws = write_idx[perm]
    jstar = jnp.searchsorted(ws, ws, side="right") - 1
    pw = perm[jstar].astype(jnp.int32)
    starts = jnp.searchsorted(
        ws, jnp.arange(NBLK + 1, dtype=jnp.int32) * BK).astype(jnp.int32)
    ws = ws.astype(jnp.int32)

    nidx3 = neg_idx.reshape(B, NCHUNK, IDX_CHUNK)

    qn, kn, lpos = _prologue(q, k)

    table0 = jnp.transpose(queue)
    table_new = _table_scatter(table0, kn, ws, pw, starts)
    lneg = _lneg_sc(table_new, qn, nidx3)
    queue_new = _untranspose(table_new)
    logits = _epilogue(lpos, lneg)
    labels = jnp.zeros((B,), dtype=jnp.int32)
    return logits, queue_new, labels
